# baseline (speedup 1.0000x reference)
import jax
import jax.numpy as jnp
import numpy as np
from jax.experimental import pallas as pl
from jax.experimental.pallas import tpu as pltpu

_B = 128
_D = 2048
_A = 1000
_ABLK = 256
_NB = (_A + _ABLK - 1) // _ABLK

_G = np.asarray(jax.random.gumbel(jax.random.key(42), (_B, _A), jnp.float32))

_NEG = np.float32(-1e30)


def _fused_body(x_ref, w_ref, b_ref, g_ref,
                act_ref, nlp_ref, ent_ref,
                m_s, s_s, t_s, bv_s, ya_s, bi_s):
    j = pl.program_id(0)

    wsum = w_ref[0:_D, :] + w_ref[_D:2 * _D, :]
    y = jnp.dot(x_ref[...], wsum, preferred_element_type=jnp.float32)
    y = y + b_ref[...]

    cols = j * _ABLK + jax.lax.broadcasted_iota(jnp.int32, (_B, _ABLK), 1)
    valid = cols < _A
    ym = jnp.where(valid, y, _NEG)

    lm = jnp.max(ym, axis=1, keepdims=True)
    le = jnp.exp(ym - lm)
    ls = jnp.sum(le, axis=1, keepdims=True)
    yv = jnp.where(valid, y, 0.0)
    lt = jnp.sum(yv * le, axis=1, keepdims=True)

    z = jnp.where(valid, y + g_ref[...], _NEG)
    lbv = jnp.max(z, axis=1, keepdims=True)
    lbi = jnp.min(jnp.where(z == lbv, cols, jnp.int32(2**30)),
                  axis=1, keepdims=True)
    lya = jnp.sum(jnp.where(cols == lbi, y, 0.0), axis=1, keepdims=True)

    @pl.when(j == 0)
    def _init():
        m_s[...] = lm
        s_s[...] = ls
        t_s[...] = lt
        bv_s[...] = lbv
        bi_s[...] = lbi
        ya_s[...] = lya

    @pl.when(j > 0)
    def _merge():
        m0 = m_s[...]
        new_m = jnp.maximum(m0, lm)
        a = jnp.exp(m0 - new_m)
        bsc = jnp.exp(lm - new_m)
        s_s[...] = s_s[...] * a + ls * bsc
        t_s[...] = t_s[...] * a + lt * bsc
        m_s[...] = new_m
        upd = lbv > bv_s[...]
        bv_s[...] = jnp.where(upd, lbv, bv_s[...])
        bi_s[...] = jnp.where(upd, lbi, bi_s[...])
        ya_s[...] = jnp.where(upd, lya, ya_s[...])

    @pl.when(j == _NB - 1)
    def _final():
        s = s_s[...]
        logz = m_s[...] + jnp.log(s)
        act_ref[...] = bi_s[...]
        nlp_ref[...] = logz - ya_s[...]
        ent_ref[...] = logz - t_s[...] / s


def kernel(x, W, b):
    g = jnp.asarray(_G)
    b2 = b.reshape(1, _A)
    act, nlp, ent = pl.pallas_call(
        _fused_body,
        grid=(_NB,),
        in_specs=[
            pl.BlockSpec((_B, _D), lambda j: (0, 0)),
            pl.BlockSpec((2 * _D, _ABLK), lambda j: (0, j)),
            pl.BlockSpec((1, _ABLK), lambda j: (0, j)),
            pl.BlockSpec((_B, _ABLK), lambda j: (0, j)),
        ],
        out_specs=[
            pl.BlockSpec((_B, 1), lambda j: (0, 0)),
            pl.BlockSpec((_B, 1), lambda j: (0, 0)),
            pl.BlockSpec((_B, 1), lambda j: (0, 0)),
        ],
        out_shape=[
            jax.ShapeDtypeStruct((_B, 1), jnp.int32),
            jax.ShapeDtypeStruct((_B, 1), jnp.float32),
            jax.ShapeDtypeStruct((_B, 1), jnp.float32),
        ],
        scratch_shapes=[
            pltpu.VMEM((_B, 1), jnp.float32),
            pltpu.VMEM((_B, 1), jnp.float32),
            pltpu.VMEM((_B, 1), jnp.float32),
            pltpu.VMEM((_B, 1), jnp.float32),
            pltpu.VMEM((_B, 1), jnp.float32),
            pltpu.VMEM((_B, 1), jnp.int32),
        ],
    )(x, W, b2, g)
    return (act.reshape(_B), nlp.reshape(_B), ent.reshape(_B))
